# Initial kernel scaffold; baseline (speedup 1.0000x reference)
import jax
import jax.numpy as jnp
from jax.experimental import pallas as pl


def kernel(src, pre_w, pre_b, gat_w, att_src, att_dst, gat_b):
    raise NotImplementedError("write your pallas kernel here")


# speedup vs baseline: 464.7021x; 464.7021x over previous
import jax
import jax.numpy as jnp
from jax.experimental import pallas as pl

_SMPL_PARENTS = (-1, 0, 0, 0, 1, 2, 3, 4, 5, 6, 7, 8, 9, 9, 9, 12, 13,
                 14, 16, 17, 18, 19, 20, 21)
_J = 24
_HID = 96
_HEADS = 3
_OUT_CH = _HID // _HEADS
_F = 512


def _encoder_block(src_ref, pre_w_ref, pre_b_ref, att_ref, gat_w_ref,
                   gat_b_ref, exp_ref, out_ref):
    pre_w = pre_w_ref[...]
    pre_b = pre_b_ref[...]
    gat_w = gat_w_ref[...]
    gat_b = gat_b_ref[...]
    att = att_ref[...]
    expand = exp_ref[...]

    xh, a_s, a_d = [], [], []
    for j in range(_J):
        x = src_ref[j]
        x = jnp.maximum(
            jnp.dot(x, pre_w, preferred_element_type=jnp.float32) + pre_b, 0.0)
        h = jnp.dot(x, gat_w, preferred_element_type=jnp.float32)
        a = jnp.dot(h, att, preferred_element_type=jnp.float32)
        xh.append(h)
        a_s.append(a[:, 0:_HEADS])
        a_d.append(a[:, _HEADS:2 * _HEADS])

    outs = []
    for j in range(_J):
        p = _SMPL_PARENTS[j]
        if p < 0:
            o = xh[j]
        else:
            al_s = a_s[j] + a_d[j]
            al_p = a_s[p] + a_d[j]
            al_s = jnp.where(al_s > 0, al_s, 0.2 * al_s)
            al_p = jnp.where(al_p > 0, al_p, 0.2 * al_p)
            m = jnp.maximum(al_s, al_p)
            es = jnp.exp(al_s - m)
            ep = jnp.exp(al_p - m)
            inv = 1.0 / (es + ep + 1e-16)
            cs = jnp.dot(es * inv, expand, preferred_element_type=jnp.float32)
            cp = jnp.dot(ep * inv, expand, preferred_element_type=jnp.float32)
            o = cs * xh[j] + cp * xh[p]
        outs.append(jnp.maximum(o + gat_b, 0.0))
    out_ref[...] = jnp.concatenate(outs, axis=1)


def kernel(src, pre_w, pre_b, gat_w, att_src, att_dst, gat_b):
    N, L, D = src.shape
    NL = N * L
    src_t = src.reshape(NL, _J, 3).transpose(1, 0, 2)
    eye = jnp.eye(_HEADS, dtype=jnp.float32)
    a_mat_s = (att_src[:, :, None] * eye[:, None, :]).reshape(_HID, _HEADS)
    a_mat_d = (att_dst[:, :, None] * eye[:, None, :]).reshape(_HID, _HEADS)
    att_mat = jnp.concatenate([a_mat_s, a_mat_d], axis=1)
    expand = jnp.repeat(eye, _OUT_CH, axis=1)

    out = pl.pallas_call(
        _encoder_block,
        grid=(NL // _F,),
        in_specs=[
            pl.BlockSpec((_J, _F, 3), lambda i: (0, i, 0)),
            pl.BlockSpec((3, _HID), lambda i: (0, 0)),
            pl.BlockSpec((1, _HID), lambda i: (0, 0)),
            pl.BlockSpec((_HID, 2 * _HEADS), lambda i: (0, 0)),
            pl.BlockSpec((_HID, _HID), lambda i: (0, 0)),
            pl.BlockSpec((1, _HID), lambda i: (0, 0)),
            pl.BlockSpec((_HEADS, _HID), lambda i: (0, 0)),
        ],
        out_specs=pl.BlockSpec((_F, _J * _HID), lambda i: (i, 0)),
        out_shape=jax.ShapeDtypeStruct((NL, _J * _HID), jnp.float32),
    )(src_t, pre_w, pre_b.reshape(1, _HID), att_mat, gat_w,
      gat_b.reshape(1, _HID), expand)
    return out.reshape(N, L, _J * _HID)
